# Initial kernel scaffold; baseline (speedup 1.0000x reference)
import jax
import jax.numpy as jnp
from jax.experimental import pallas as pl


def kernel(x, Wg, bg, W1, W2, W3):
    raise NotImplementedError("write your pallas kernel here")


# speedup vs baseline: 3.5910x; 3.5910x over previous
import functools

import jax
import jax.numpy as jnp
from jax.experimental import pallas as pl
from jax.experimental.pallas import tpu as pltpu

_HIDDEN = 768
_FFN = 1024
_E = 8
_JITTER = 0.01
_NEG = -1e30
_CHUNK = 512


def _gelu_exact(v):
    return 0.5 * v * (1.0 + jax.lax.erf(v * 0.7071067811865476))


def _router(logits):
    m = jnp.max(logits, axis=1, keepdims=True)
    ex = jnp.exp(logits - m)
    scores = ex / jnp.sum(ex, axis=1, keepdims=True)
    iota = jax.lax.broadcasted_iota(jnp.int32, scores.shape, 1)
    max1 = jnp.max(scores, axis=1, keepdims=True)
    factor = jnp.maximum(jnp.abs(scores), max1)
    mask1 = ((max1 - scores) / factor) > (2.0 * _JITTER)
    mg1 = jnp.where(mask1, _NEG, scores)
    m1 = jnp.max(mg1, axis=1, keepdims=True)
    e1 = jnp.exp(mg1 - m1)
    p1 = e1 / jnp.sum(e1, axis=1, keepdims=True)
    mult1 = jnp.max(p1, axis=1, keepdims=True)
    sel1 = jnp.min(jnp.where(scores == max1, iota, _E), axis=1, keepdims=True)
    ms = jnp.where(iota == sel1, _NEG, scores)
    max2 = jnp.max(ms, axis=1, keepdims=True)
    factor2 = jnp.maximum(jnp.abs(scores), max2)
    mask2 = ((max2 - scores) / factor2) > (2.0 * _JITTER)
    mg2 = jnp.where(mask2, _NEG, ms)
    m2 = jnp.max(mg2, axis=1, keepdims=True)
    e2 = jnp.exp(mg2 - m2)
    p2 = e2 / jnp.sum(e2, axis=1, keepdims=True)
    mult2 = jnp.max(p2, axis=1, keepdims=True)
    sel2 = jnp.min(jnp.where(ms == max2, iota, _E), axis=1, keepdims=True)
    return mult1, mult2, sel1, sel2


def _moe_body(xf_ref, wg_ref, bg_ref, w1_ref, w2_ref, w3_ref, out_ref,
              wf_ref, si_ref):
    e = pl.program_id(0)

    @pl.when(e == 0)
    def _():
        logits = jax.lax.dot_general(
            xf_ref[...], wg_ref[...], (((1,), (1,)), ((), ())),
            preferred_element_type=jnp.float32) + bg_ref[...]
        mult1, mult2, sel1, sel2 = _router(logits)
        wf_ref[:, 0:1] = mult1
        wf_ref[:, 1:2] = mult2
        si_ref[:, 0:1] = sel1
        si_ref[:, 1:2] = sel2
        out_ref[...] = jnp.zeros_like(out_ref)

    we = (jnp.where(si_ref[:, 0:1] == e, wf_ref[:, 0:1], 0.0)
          + jnp.where(si_ref[:, 1:2] == e, wf_ref[:, 1:2], 0.0))
    w1e = w1_ref[0]
    w2e = w2_ref[0]
    w3e = w3_ref[0]
    n = xf_ref.shape[0]
    for c in range(0, n, _CHUNK):
        xc = xf_ref[c:c + _CHUNK, :]
        h1 = jax.lax.dot_general(xc, w1e, (((1,), (1,)), ((), ())),
                                 preferred_element_type=jnp.float32)
        h3 = jax.lax.dot_general(xc, w3e, (((1,), (1,)), ((), ())),
                                 preferred_element_type=jnp.float32)
        hh = _gelu_exact(h1) * h3
        oe = jax.lax.dot_general(hh, w2e, (((1,), (1,)), ((), ())),
                                 preferred_element_type=jnp.float32)
        out_ref[c:c + _CHUNK, :] += oe * we[c:c + _CHUNK, :]


@jax.jit
def kernel(x, Wg, bg, W1, W2, W3):
    b, s, h = x.shape
    xf = x.reshape(s, h)
    bg2 = bg.reshape(1, _E)
    out = pl.pallas_call(
        _moe_body,
        grid=(_E,),
        in_specs=[
            pl.BlockSpec((s, h), lambda e: (0, 0)),
            pl.BlockSpec((_E, h), lambda e: (0, 0)),
            pl.BlockSpec((1, _E), lambda e: (0, 0)),
            pl.BlockSpec((1, _FFN, h), lambda e: (e, 0, 0)),
            pl.BlockSpec((1, h, _FFN), lambda e: (e, 0, 0)),
            pl.BlockSpec((1, _FFN, h), lambda e: (e, 0, 0)),
        ],
        out_specs=pl.BlockSpec((s, h), lambda e: (0, 0)),
        out_shape=jax.ShapeDtypeStruct((s, h), jnp.float32),
        scratch_shapes=[
            pltpu.VMEM((s, 8), jnp.float32),
            pltpu.VMEM((s, 8), jnp.int32),
        ],
    )(xf, Wg, bg2, W1, W2, W3)
    return out.reshape(b, s, h)


# speedup vs baseline: 3.5921x; 1.0003x over previous
import functools

import jax
import jax.numpy as jnp
from jax.experimental import pallas as pl
from jax.experimental.pallas import tpu as pltpu

_HIDDEN = 768
_FFN = 1024
_E = 8
_JITTER = 0.01
_NEG = -1e30
_CHUNK = 512


def _gelu_exact(v):
    return 0.5 * v * (1.0 + jax.lax.erf(v * 0.7071067811865476))


def _router(logits):
    m = jnp.max(logits, axis=1, keepdims=True)
    ex = jnp.exp(logits - m)
    scores = ex / jnp.sum(ex, axis=1, keepdims=True)
    iota = jax.lax.broadcasted_iota(jnp.int32, scores.shape, 1)
    max1 = jnp.max(scores, axis=1, keepdims=True)
    factor = jnp.maximum(jnp.abs(scores), max1)
    mask1 = ((max1 - scores) / factor) > (2.0 * _JITTER)
    mg1 = jnp.where(mask1, _NEG, scores)
    m1 = jnp.max(mg1, axis=1, keepdims=True)
    e1 = jnp.exp(mg1 - m1)
    p1 = e1 / jnp.sum(e1, axis=1, keepdims=True)
    mult1 = jnp.max(p1, axis=1, keepdims=True)
    sel1 = jnp.min(jnp.where(scores == max1, iota, _E), axis=1, keepdims=True)
    ms = jnp.where(iota == sel1, _NEG, scores)
    max2 = jnp.max(ms, axis=1, keepdims=True)
    factor2 = jnp.maximum(jnp.abs(scores), max2)
    mask2 = ((max2 - scores) / factor2) > (2.0 * _JITTER)
    mg2 = jnp.where(mask2, _NEG, ms)
    m2 = jnp.max(mg2, axis=1, keepdims=True)
    e2 = jnp.exp(mg2 - m2)
    p2 = e2 / jnp.sum(e2, axis=1, keepdims=True)
    mult2 = jnp.max(p2, axis=1, keepdims=True)
    sel2 = jnp.min(jnp.where(ms == max2, iota, _E), axis=1, keepdims=True)
    return mult1, mult2, sel1, sel2


def _moe_body(xf_ref, wg_ref, bg_ref, w1_ref, w2_ref, w3_ref, out_ref,
              wf_ref, si_ref):
    e = pl.program_id(0)

    @pl.when(e == 0)
    def _():
        logits = jax.lax.dot_general(
            xf_ref[...], wg_ref[...], (((1,), (1,)), ((), ())),
            preferred_element_type=jnp.float32) + bg_ref[...]
        mult1, mult2, sel1, sel2 = _router(logits)
        wf_ref[:, 0:1] = mult1
        wf_ref[:, 1:2] = mult2
        si_ref[:, 0:1] = sel1
        si_ref[:, 1:2] = sel2
        out_ref[...] = jnp.zeros_like(out_ref)

    we = (jnp.where(si_ref[:, 0:1] == e, wf_ref[:, 0:1], 0.0)
          + jnp.where(si_ref[:, 1:2] == e, wf_ref[:, 1:2], 0.0))
    w1e = w1_ref[0].astype(jnp.bfloat16)
    w2e = w2_ref[0].astype(jnp.bfloat16)
    w3e = w3_ref[0].astype(jnp.bfloat16)
    n = xf_ref.shape[0]
    for c in range(0, n, _CHUNK):
        xc = xf_ref[c:c + _CHUNK, :].astype(jnp.bfloat16)
        h1 = jax.lax.dot_general(xc, w1e, (((1,), (1,)), ((), ())),
                                 preferred_element_type=jnp.float32)
        h3 = jax.lax.dot_general(xc, w3e, (((1,), (1,)), ((), ())),
                                 preferred_element_type=jnp.float32)
        hh = (_gelu_exact(h1) * h3).astype(jnp.bfloat16)
        oe = jax.lax.dot_general(hh, w2e, (((1,), (1,)), ((), ())),
                                 preferred_element_type=jnp.float32)
        out_ref[c:c + _CHUNK, :] += oe * we[c:c + _CHUNK, :]


@jax.jit
def kernel(x, Wg, bg, W1, W2, W3):
    b, s, h = x.shape
    xf = x.reshape(s, h)
    bg2 = bg.reshape(1, _E)
    out = pl.pallas_call(
        _moe_body,
        grid=(_E,),
        in_specs=[
            pl.BlockSpec((s, h), lambda e: (0, 0)),
            pl.BlockSpec((_E, h), lambda e: (0, 0)),
            pl.BlockSpec((1, _E), lambda e: (0, 0)),
            pl.BlockSpec((1, _FFN, h), lambda e: (e, 0, 0)),
            pl.BlockSpec((1, h, _FFN), lambda e: (e, 0, 0)),
            pl.BlockSpec((1, _FFN, h), lambda e: (e, 0, 0)),
        ],
        out_specs=pl.BlockSpec((s, h), lambda e: (0, 0)),
        out_shape=jax.ShapeDtypeStruct((s, h), jnp.float32),
        scratch_shapes=[
            pltpu.VMEM((s, 8), jnp.float32),
            pltpu.VMEM((s, 8), jnp.int32),
        ],
    )(xf, Wg, bg2, W1, W2, W3)
    return out.reshape(b, s, h)
